# baseline (speedup 1.0000x reference)
import functools

import numpy as np

import jax
import jax.numpy as jnp
from jax import lax
from jax.experimental import pallas as pl
from jax.experimental.pallas import tpu as pltpu
from jax.experimental.pallas import tpu_sc as plsc

N = 10000
E = 320000
D = 128
NC = 2
NS = 16
NW = NC * NS
CHUNK = 32
EPW = ((E + NW * 256 - 1) // (NW * 256)) * 256
CHUNKS = EPW // CHUNK
E_PAD = NW * EPW
N_PAD = 10240
ROWS_PER_TILE = N_PAD // NS


NBUF = 10
GROUP = 32
GROUP_ROWS = GROUP * CHUNK // 128


def _sc_body(feat_hbm, sd_hbm, w_hbm, out_hbm,
             sd_v, w_v,
             gbuf0, gbuf1, gbuf2, gbuf3, gbuf4, gbuf5, gbuf6, gbuf7,
             gbuf8, gbuf9,
             srcb0, srcb1, srcb2, srcb3, srcb4, srcb5, srcb6, srcb7,
             srcb8, srcb9,
             dstb0, dstb1, dstb2, dstb3, dstb4, dstb5, dstb6, dstb7,
             dstb8, dstb9,
             gsem0, gsem1, gsem2, gsem3, gsem4, gsem5, gsem6, gsem7,
             gsem8, gsem9,
             ssem0, ssem1, isem,
             acc_sh):
    gbuf = (gbuf0, gbuf1, gbuf2, gbuf3, gbuf4, gbuf5, gbuf6, gbuf7,
            gbuf8, gbuf9)
    srcb = (srcb0, srcb1, srcb2, srcb3, srcb4, srcb5, srcb6, srcb7,
            srcb8, srcb9)
    dstb = (dstb0, dstb1, dstb2, dstb3, dstb4, dstb5, dstb6, dstb7,
            dstb8, dstb9)
    gsem = (gsem0, gsem1, gsem2, gsem3, gsem4, gsem5, gsem6, gsem7,
            gsem8, gsem9)
    ssem = (ssem0, ssem1)

    cid = lax.axis_index("c")
    sid = lax.axis_index("s")
    wid = sid * NC + cid

    slab = pl.ds(sid * ROWS_PER_TILE, ROWS_PER_TILE)

    def zfill(i, c):
        gbuf0[i // (D // 16), pl.ds((i % (D // 16)) * 16, 16)] = (
            jnp.zeros((16,), jnp.float32)
        )
        return c

    lax.fori_loop(0, CHUNK * D // 16, zfill, 0, unroll=False)

    def zcopy(k, c):
        pltpu.sync_copy(
            gbuf0, acc_sh.at[pl.ds(sid * ROWS_PER_TILE + k * CHUNK, CHUNK)]
        )
        return c

    lax.fori_loop(0, ROWS_PER_TILE // CHUNK, zcopy, 0, unroll=False)
    plsc.subcore_barrier()

    def istart(q):
        sl_h = pl.ds(q * GROUP_ROWS, GROUP_ROWS)
        sl_v = pl.ds((q % 2) * GROUP_ROWS, GROUP_ROWS)
        pltpu.async_copy(sd_hbm.at[wid, sl_h], sd_v.at[sl_v], isem)
        pltpu.async_copy(w_hbm.at[wid, sl_h], w_v.at[sl_v], isem)

    def iwait(q):
        sl_h = pl.ds(q * GROUP_ROWS, GROUP_ROWS)
        sl_v = pl.ds((q % 2) * GROUP_ROWS, GROUP_ROWS)
        pltpu.make_async_copy(sd_hbm.at[wid, sl_h], sd_v.at[sl_v],
                              isem).wait()
        pltpu.make_async_copy(w_hbm.at[wid, sl_h], w_v.at[sl_v],
                              isem).wait()

    def unpack(j, b):
        for g in range(CHUNK // 16):
            off = (j % (2 * GROUP)) * CHUNK + g * 16
            sl = pl.ds(g * 16, 16)
            v = sd_v[off // 128, pl.ds(off % 128, 16)]
            srcb[b][sl] = v & 0x3FFF
            dstb[b][sl] = v >> 14

    def gstart(j, b):
        pltpu.async_copy(feat_hbm.at[srcb[b]], gbuf[b], gsem[b])

    def gwait(j, b):
        pltpu.make_async_copy(feat_hbm.at[srcb[b]], gbuf[b], gsem[b]).wait()

    def cstart(b, s):
        pltpu.async_copy(gbuf[b], acc_sh.at[dstb[b]], ssem[s], add=True)

    def cwait(b, s):
        pltpu.make_async_copy(gbuf[b], acc_sh.at[dstb[b]], ssem[s]).wait()

    def scale(j, b, s):
        def group_body(g, c):
            off = (j % (2 * GROUP)) * CHUNK + g * 16
            w_vec = w_v[off // 128, pl.ds(off % 128, 16)]
            for rr in range(16):
                row = g * 16 + rr
                ws = w_vec[rr]
                for c8 in range(D // 16):
                    sl = pl.ds(c8 * 16, 16)
                    gbuf[b][row, sl] = gbuf[b][row, sl] * ws
            return c

        lax.fori_loop(0, CHUNK // 16, group_body, 0, unroll=False)

    istart(0)
    iwait(0)
    for b in range(NBUF - 1):
        unpack(b, b)
        gstart(b, b)

    def pipe_body(jj, carry):
        for b in range(NBUF):
            j = jj * NBUF + b
            s = b % 2
            gwait(j, b)

            jm = j % GROUP

            @pl.when(jnp.logical_and(jm == GROUP // 4, j < CHUNKS - GROUP))
            def _():
                istart(j // GROUP + 1)

            @pl.when(jnp.logical_and(jm == 3 * GROUP // 4,
                                     j < CHUNKS - GROUP))
            def _():
                iwait(j // GROUP + 1)

            scale(j, b, s)
            cstart(b, s)
            bn = (b + NBUF - 1) % NBUF
            sn = (s + 1) % 2
            jn = j + NBUF - 1
            if b == 0:
                @pl.when(jj > 0)
                def _():
                    cwait(bn, sn)

                unpack(jn, bn)
                gstart(jn, bn)
            else:
                cwait(bn, sn)

                @pl.when(jn < CHUNKS)
                def _():
                    unpack(jn, bn)
                    gstart(jn, bn)
        return carry

    lax.fori_loop(0, CHUNKS // NBUF, pipe_body, 0, unroll=False)
    cwait(NBUF - 1, (NBUF - 1) % 2)
    plsc.subcore_barrier()

    pltpu.sync_copy(acc_sh.at[slab], out_hbm.at[cid, slab])


def _make_sc_kernel():
    mesh = plsc.VectorSubcoreMesh(core_axis_name="c", subcore_axis_name="s")
    return pl.kernel(
        _sc_body,
        out_type=jax.ShapeDtypeStruct((NC, N_PAD, D), jnp.float32),
        mesh=mesh,
        compiler_params=pltpu.CompilerParams(needs_layout_passes=False),
        scratch_types=[
            pltpu.VMEM((2 * GROUP_ROWS, 128), jnp.int32),
            pltpu.VMEM((2 * GROUP_ROWS, 128), jnp.float32),
        ]
        + [pltpu.VMEM((CHUNK, D), jnp.float32) for _ in range(NBUF)]
        + [pltpu.VMEM((CHUNK,), jnp.int32) for _ in range(2 * NBUF)]
        + [pltpu.SemaphoreType.DMA for _ in range(NBUF + 3)]
        + [
            pltpu.VMEM_SHARED((N_PAD, D), jnp.float32),
        ],
    )


def _mm_body(p_ref, w_ref, b_ref, o_ref):
    x = p_ref[0] + p_ref[1]
    o_ref[...] = (
        jnp.dot(x, w_ref[...], preferred_element_type=jnp.float32) + b_ref[...]
    )


MM_BLOCK = 400


def _make_mm_kernel():
    return pl.pallas_call(
        _mm_body,
        grid=(N // MM_BLOCK,),
        in_specs=[
            pl.BlockSpec((NC, MM_BLOCK, D), lambda i: (0, i, 0)),
            pl.BlockSpec((D, D), lambda i: (0, 0)),
            pl.BlockSpec((1, D), lambda i: (0, 0)),
        ],
        out_specs=pl.BlockSpec((MM_BLOCK, D), lambda i: (i, 0)),
        out_shape=jax.ShapeDtypeStruct((N, D), jnp.float32),
    )


def kernel(features, edge_index, edge_weight, W, bias):
    src = edge_index[0]
    dst = edge_index[1]

    pad = E_PAD - E
    if pad:
        fill = (jnp.arange(pad, dtype=jnp.int32) * 37) % N
        src = jnp.concatenate([src, fill])
        dst = jnp.concatenate([dst, fill])
        edge_weight = jnp.concatenate(
            [edge_weight, jnp.zeros((pad,), jnp.float32)]
        )

    sd = (dst << 14) | src
    sd_r = sd.reshape(NW, EPW // 128, 128)
    w_r = edge_weight.reshape(NW, EPW // 128, 128)

    partials = _make_sc_kernel()(features, sd_r, w_r)
    out = _make_mm_kernel()(partials, W, bias.reshape(1, D))
    return out


# speedup vs baseline: 1.1198x; 1.1198x over previous
import functools

import numpy as np

import jax
import jax.numpy as jnp
from jax import lax
from jax.experimental import pallas as pl
from jax.experimental.pallas import tpu as pltpu
from jax.experimental.pallas import tpu_sc as plsc

N = 10000
E = 320000
D = 128
NC = 2
NS = 16
NW = NC * NS
CHUNK = 32
EPW = ((E + NW * 256 - 1) // (NW * 256)) * 256
CHUNKS = EPW // CHUNK
E_PAD = NW * EPW
N_PAD = 10240
ROWS_PER_TILE = N_PAD // NS


NBUF = 8
GROUP = 32
GROUP_ROWS = GROUP * CHUNK // 128


def _sc_body(feat_hbm, sd_hbm, w_hbm, out_hbm,
             sd_v, w_v,
             gbuf0, gbuf1, gbuf2, gbuf3, gbuf4, gbuf5, gbuf6, gbuf7,
             sbuf0, sbuf1,
             srcb0, srcb1, srcb2, srcb3, srcb4, srcb5, srcb6, srcb7,
             dstb0, dstb1, dstb2, dstb3, dstb4, dstb5, dstb6, dstb7,
             gsem0, gsem1, gsem2, gsem3, gsem4, gsem5, gsem6, gsem7,
             ssem0, ssem1, isem,
             acc_sh):
    gbuf = (gbuf0, gbuf1, gbuf2, gbuf3, gbuf4, gbuf5, gbuf6, gbuf7)
    sbuf = (sbuf0, sbuf1)
    srcb = (srcb0, srcb1, srcb2, srcb3, srcb4, srcb5, srcb6, srcb7)
    dstb = (dstb0, dstb1, dstb2, dstb3, dstb4, dstb5, dstb6, dstb7)
    gsem = (gsem0, gsem1, gsem2, gsem3, gsem4, gsem5, gsem6, gsem7)
    ssem = (ssem0, ssem1)

    cid = lax.axis_index("c")
    sid = lax.axis_index("s")
    wid = sid * NC + cid

    slab = pl.ds(sid * ROWS_PER_TILE, ROWS_PER_TILE)

    def zfill(i, c):
        gbuf0[i // (D // 16), pl.ds((i % (D // 16)) * 16, 16)] = (
            jnp.zeros((16,), jnp.float32)
        )
        return c

    lax.fori_loop(0, CHUNK * D // 16, zfill, 0, unroll=False)

    def zcopy(k, c):
        pltpu.sync_copy(
            gbuf0, acc_sh.at[pl.ds(sid * ROWS_PER_TILE + k * CHUNK, CHUNK)]
        )
        return c

    lax.fori_loop(0, ROWS_PER_TILE // CHUNK, zcopy, 0, unroll=False)
    plsc.subcore_barrier()

    def istart(q):
        sl_h = pl.ds(q * GROUP_ROWS, GROUP_ROWS)
        sl_v = pl.ds((q % 2) * GROUP_ROWS, GROUP_ROWS)
        pltpu.async_copy(sd_hbm.at[wid, sl_h], sd_v.at[sl_v], isem)
        pltpu.async_copy(w_hbm.at[wid, sl_h], w_v.at[sl_v], isem)

    def iwait(q):
        sl_h = pl.ds(q * GROUP_ROWS, GROUP_ROWS)
        sl_v = pl.ds((q % 2) * GROUP_ROWS, GROUP_ROWS)
        pltpu.make_async_copy(sd_hbm.at[wid, sl_h], sd_v.at[sl_v],
                              isem).wait()
        pltpu.make_async_copy(w_hbm.at[wid, sl_h], w_v.at[sl_v],
                              isem).wait()

    def unpack(j, b):
        for g in range(CHUNK // 16):
            off = (j % (2 * GROUP)) * CHUNK + g * 16
            sl = pl.ds(g * 16, 16)
            v = sd_v[off // 128, pl.ds(off % 128, 16)]
            srcb[b][sl] = v & 0x3FFF
            dstb[b][sl] = v >> 14

    def gstart(j, b):
        pltpu.async_copy(feat_hbm.at[srcb[b]], gbuf[b], gsem[b])

    def gwait(j, b):
        pltpu.make_async_copy(feat_hbm.at[srcb[b]], gbuf[b], gsem[b]).wait()

    def cstart(b, s):
        pltpu.async_copy(sbuf[s], acc_sh.at[dstb[b]], ssem[s], add=True)

    def cwait(b, s):
        pltpu.make_async_copy(sbuf[s], acc_sh.at[dstb[b]], ssem[s]).wait()

    def scale(j, b, s):
        def group_body(g, c):
            off = (j % (2 * GROUP)) * CHUNK + g * 16
            w_vec = w_v[off // 128, pl.ds(off % 128, 16)]
            for rr in range(16):
                row = g * 16 + rr
                ws = w_vec[rr]
                for c8 in range(D // 16):
                    sl = pl.ds(c8 * 16, 16)
                    sbuf[s][row, sl] = gbuf[b][row, sl] * ws
            return c

        lax.fori_loop(0, CHUNK // 16, group_body, 0, unroll=False)

    istart(0)
    iwait(0)
    for b in range(NBUF - 1):
        unpack(b, b)
        gstart(b, b)

    def pipe_body(jj, carry):
        for b in range(NBUF):
            j = jj * NBUF + b
            s = b % 2
            gwait(j, b)

            jm = j % GROUP

            @pl.when(jnp.logical_and(jm == GROUP // 4, j < CHUNKS - GROUP))
            def _():
                istart(j // GROUP + 1)

            @pl.when(jnp.logical_and(jm == 3 * GROUP // 4,
                                     j < CHUNKS - GROUP))
            def _():
                iwait(j // GROUP + 1)

            scale(j, b, s)
            cstart(b, s)
            bn = (b + NBUF - 1) % NBUF
            sn = (s + 1) % 2
            jn = j + NBUF - 1
            if b == 0:
                @pl.when(jj > 0)
                def _():
                    cwait(bn, sn)

                unpack(jn, bn)
                gstart(jn, bn)
            else:
                cwait(bn, sn)

                @pl.when(jn < CHUNKS)
                def _():
                    unpack(jn, bn)
                    gstart(jn, bn)
        return carry

    lax.fori_loop(0, CHUNKS // NBUF, pipe_body, 0, unroll=False)
    cwait(NBUF - 1, (NBUF - 1) % 2)
    plsc.subcore_barrier()

    pltpu.sync_copy(acc_sh.at[slab], out_hbm.at[cid, slab])


def _make_sc_kernel():
    mesh = plsc.VectorSubcoreMesh(core_axis_name="c", subcore_axis_name="s")
    return pl.kernel(
        _sc_body,
        out_type=jax.ShapeDtypeStruct((NC, N_PAD, D), jnp.float32),
        mesh=mesh,
        compiler_params=pltpu.CompilerParams(needs_layout_passes=False),
        scratch_types=[
            pltpu.VMEM((2 * GROUP_ROWS, 128), jnp.int32),
            pltpu.VMEM((2 * GROUP_ROWS, 128), jnp.float32),
        ]
        + [pltpu.VMEM((CHUNK, D), jnp.float32) for _ in range(NBUF)]
        + [pltpu.VMEM((CHUNK, D), jnp.float32) for _ in range(2)]
        + [pltpu.VMEM((CHUNK,), jnp.int32) for _ in range(2 * NBUF)]
        + [pltpu.SemaphoreType.DMA for _ in range(NBUF + 3)]
        + [
            pltpu.VMEM_SHARED((N_PAD, D), jnp.float32),
        ],
    )


def _mm_body(p_ref, w_ref, b_ref, o_ref):
    x = p_ref[0] + p_ref[1]
    o_ref[...] = (
        jnp.dot(x, w_ref[...], preferred_element_type=jnp.float32) + b_ref[...]
    )


MM_BLOCK = 2000


def _make_mm_kernel():
    return pl.pallas_call(
        _mm_body,
        grid=(N // MM_BLOCK,),
        in_specs=[
            pl.BlockSpec((NC, MM_BLOCK, D), lambda i: (0, i, 0)),
            pl.BlockSpec((D, D), lambda i: (0, 0)),
            pl.BlockSpec((1, D), lambda i: (0, 0)),
        ],
        out_specs=pl.BlockSpec((MM_BLOCK, D), lambda i: (i, 0)),
        out_shape=jax.ShapeDtypeStruct((N, D), jnp.float32),
    )


def kernel(features, edge_index, edge_weight, W, bias):
    src = edge_index[0]
    dst = edge_index[1]

    pad = E_PAD - E
    if pad:
        fill = (jnp.arange(pad, dtype=jnp.int32) * 37) % N
        src = jnp.concatenate([src, fill])
        dst = jnp.concatenate([dst, fill])
        edge_weight = jnp.concatenate(
            [edge_weight, jnp.zeros((pad,), jnp.float32)]
        )

    sd = (dst << 14) | src
    sd_r = sd.reshape(NW, EPW // 128, 128)
    w_r = edge_weight.reshape(NW, EPW // 128, 128)

    partials = _make_sc_kernel()(features, sd_r, w_r)
    out = _make_mm_kernel()(partials, W, bias.reshape(1, D))
    return out


# speedup vs baseline: 1.1344x; 1.0130x over previous
import functools

import numpy as np

import jax
import jax.numpy as jnp
from jax import lax
from jax.experimental import pallas as pl
from jax.experimental.pallas import tpu as pltpu
from jax.experimental.pallas import tpu_sc as plsc

N = 10000
E = 320000
D = 128
NC = 2
NS = 16
NW = NC * NS
CHUNK = 32
EPW = ((E + NW * 256 - 1) // (NW * 256)) * 256
CHUNKS = EPW // CHUNK
E_PAD = NW * EPW
N_PAD = 10240
ROWS_PER_TILE = N_PAD // NS


NBUF = 8
GROUP = 32
GROUP_ROWS = GROUP * CHUNK // 128


def _sc_body(feat_hbm, sd_hbm, w_hbm, out_hbm,
             sd_v, w_v,
             gbuf0, gbuf1, gbuf2, gbuf3, gbuf4, gbuf5, gbuf6, gbuf7,
             sbuf0, sbuf1,
             srcb0, srcb1, srcb2, srcb3, srcb4, srcb5, srcb6, srcb7,
             dstb0, dstb1, dstb2, dstb3, dstb4, dstb5, dstb6, dstb7,
             gsem0, gsem1, gsem2, gsem3, gsem4, gsem5, gsem6, gsem7,
             ssem0, ssem1, isem,
             acc_sh):
    gbuf = (gbuf0, gbuf1, gbuf2, gbuf3, gbuf4, gbuf5, gbuf6, gbuf7)
    sbuf = (sbuf0, sbuf1)
    srcb = (srcb0, srcb1, srcb2, srcb3, srcb4, srcb5, srcb6, srcb7)
    dstb = (dstb0, dstb1, dstb2, dstb3, dstb4, dstb5, dstb6, dstb7)
    gsem = (gsem0, gsem1, gsem2, gsem3, gsem4, gsem5, gsem6, gsem7)
    ssem = (ssem0, ssem1)

    cid = lax.axis_index("c")
    sid = lax.axis_index("s")
    wid = sid * NC + cid

    slab = pl.ds(sid * ROWS_PER_TILE, ROWS_PER_TILE)

    def zfill(i, c):
        gbuf0[i // (D // 16), pl.ds((i % (D // 16)) * 16, 16)] = (
            jnp.zeros((16,), jnp.float32)
        )
        return c

    lax.fori_loop(0, CHUNK * D // 16, zfill, 0, unroll=False)

    def zcopy(k, c):
        pltpu.sync_copy(
            gbuf0, acc_sh.at[pl.ds(sid * ROWS_PER_TILE + k * CHUNK, CHUNK)]
        )
        return c

    lax.fori_loop(0, ROWS_PER_TILE // CHUNK, zcopy, 0, unroll=False)
    plsc.subcore_barrier()

    def istart(q):
        sl_h = pl.ds(q * GROUP_ROWS, GROUP_ROWS)
        sl_v = pl.ds((q % 2) * GROUP_ROWS, GROUP_ROWS)
        pltpu.async_copy(sd_hbm.at[wid, sl_h], sd_v.at[sl_v], isem)
        pltpu.async_copy(w_hbm.at[wid, sl_h], w_v.at[sl_v], isem)

    def iwait(q):
        sl_h = pl.ds(q * GROUP_ROWS, GROUP_ROWS)
        sl_v = pl.ds((q % 2) * GROUP_ROWS, GROUP_ROWS)
        pltpu.make_async_copy(sd_hbm.at[wid, sl_h], sd_v.at[sl_v],
                              isem).wait()
        pltpu.make_async_copy(w_hbm.at[wid, sl_h], w_v.at[sl_v],
                              isem).wait()

    def unpack(j, b):
        for g in range(CHUNK // 16):
            off = (j % (2 * GROUP)) * CHUNK + g * 16
            sl = pl.ds(g * 16, 16)
            v = sd_v[off // 128, pl.ds(off % 128, 16)]
            srcb[b][sl] = v & 0x3FFF
            dstb[b][sl] = v >> 14

    def gstart(j, b):
        pltpu.async_copy(feat_hbm.at[srcb[b]], gbuf[b], gsem[b])

    def gwait(j, b):
        pltpu.make_async_copy(feat_hbm.at[srcb[b]], gbuf[b], gsem[b]).wait()

    def cstart(b, s):
        pltpu.async_copy(sbuf[s], acc_sh.at[dstb[b]], ssem[s], add=True)

    def cwait(b, s):
        pltpu.make_async_copy(sbuf[s], acc_sh.at[dstb[b]], ssem[s]).wait()

    def scale(j, b, s):
        def group_body(g, c):
            off = (j % (2 * GROUP)) * CHUNK + g * 16
            w_vec = w_v[off // 128, pl.ds(off % 128, 16)]
            for rr in range(16):
                row = g * 16 + rr
                ws = w_vec[rr]
                for c8 in range(D // 16):
                    sl = pl.ds(c8 * 16, 16)
                    sbuf[s][row, sl] = gbuf[b][row, sl] * ws
            return c

        lax.fori_loop(0, CHUNK // 16, group_body, 0, unroll=False)

    istart(0)
    iwait(0)
    for b in range(NBUF - 1):
        unpack(b, b)
        gstart(b, b)

    def pipe_body(jj, carry):
        for b in range(NBUF):
            j = jj * NBUF + b
            s = b % 2
            gwait(j, b)

            jm = j % GROUP

            @pl.when(jnp.logical_and(jm == GROUP // 4, j < CHUNKS - GROUP))
            def _():
                istart(j // GROUP + 1)

            @pl.when(jnp.logical_and(jm == 3 * GROUP // 4,
                                     j < CHUNKS - GROUP))
            def _():
                iwait(j // GROUP + 1)

            scale(j, b, s)
            cstart(b, s)
            bn = (b + NBUF - 1) % NBUF
            sn = (s + 1) % 2
            jn = j + NBUF - 1
            if b == 0:
                @pl.when(jj > 0)
                def _():
                    cwait(bn, sn)

                unpack(jn, bn)
                gstart(jn, bn)
            else:
                cwait(bn, sn)

                @pl.when(jn < CHUNKS)
                def _():
                    unpack(jn, bn)
                    gstart(jn, bn)
        return carry

    lax.fori_loop(0, CHUNKS // NBUF, pipe_body, 0, unroll=False)
    cwait(NBUF - 1, (NBUF - 1) % 2)
    plsc.subcore_barrier()

    pltpu.sync_copy(acc_sh.at[slab], out_hbm.at[cid, slab])


def _make_sc_kernel():
    mesh = plsc.VectorSubcoreMesh(core_axis_name="c", subcore_axis_name="s")
    return pl.kernel(
        _sc_body,
        out_type=jax.ShapeDtypeStruct((NC, N_PAD, D), jnp.float32),
        mesh=mesh,
        compiler_params=pltpu.CompilerParams(needs_layout_passes=False),
        scratch_types=[
            pltpu.VMEM((2 * GROUP_ROWS, 128), jnp.int32),
            pltpu.VMEM((2 * GROUP_ROWS, 128), jnp.float32),
        ]
        + [pltpu.VMEM((CHUNK, D), jnp.float32) for _ in range(NBUF)]
        + [pltpu.VMEM((CHUNK, D), jnp.float32) for _ in range(2)]
        + [pltpu.VMEM((CHUNK,), jnp.int32) for _ in range(2 * NBUF)]
        + [pltpu.SemaphoreType.DMA for _ in range(NBUF + 3)]
        + [
            pltpu.VMEM_SHARED((N_PAD, D), jnp.float32),
        ],
    )


def _mm_body(p_ref, w_ref, b_ref, o_ref):
    x = p_ref[0] + p_ref[1]
    o_ref[...] = (
        jnp.dot(x, w_ref[...], preferred_element_type=jnp.float32) + b_ref[...]
    )


MM_BLOCK = 5000


def _make_mm_kernel():
    return pl.pallas_call(
        _mm_body,
        grid=(N // MM_BLOCK,),
        in_specs=[
            pl.BlockSpec((NC, MM_BLOCK, D), lambda i: (0, i, 0)),
            pl.BlockSpec((D, D), lambda i: (0, 0)),
            pl.BlockSpec((1, D), lambda i: (0, 0)),
        ],
        out_specs=pl.BlockSpec((MM_BLOCK, D), lambda i: (i, 0)),
        out_shape=jax.ShapeDtypeStruct((N, D), jnp.float32),
    )


def kernel(features, edge_index, edge_weight, W, bias):
    src = edge_index[0]
    dst = edge_index[1]

    pad = E_PAD - E
    if pad:
        fill = (jnp.arange(pad, dtype=jnp.int32) * 37) % N
        src = jnp.concatenate([src, fill])
        dst = jnp.concatenate([dst, fill])
        edge_weight = jnp.concatenate(
            [edge_weight, jnp.zeros((pad,), jnp.float32)]
        )

    sd = (dst << 14) | src
    sd_r = sd.reshape(NW, EPW // 128, 128)
    w_r = edge_weight.reshape(NW, EPW // 128, 128)

    partials = _make_sc_kernel()(features, sd_r, w_r)
    out = _make_mm_kernel()(partials, W, bias.reshape(1, D))
    return out
